# baseline (speedup 1.0000x reference)
import functools

import jax
import jax.numpy as jnp
from jax import lax
from jax.experimental import pallas as pl
from jax.experimental.pallas import tpu as pltpu
from jax.experimental.pallas import tpu_sc as plsc

N_NODES = 100000
D = 256
B = 16384

_NC = 2
_NS = 16
_NW = _NC * _NS
_K = 128

_G_BPW = B // _NW
_G_NCH = _G_BPW // _K

_gather_mesh = plsc.VectorSubcoreMesh(core_axis_name="c", subcore_axis_name="s")


@functools.partial(
    pl.kernel,
    out_type=jax.ShapeDtypeStruct((B, D), jnp.float32),
    mesh=_gather_mesh,
    scratch_types=[
        pltpu.VMEM((_G_NCH, _K), jnp.int32),
        pltpu.VMEM((_K, D), jnp.float32),
        pltpu.VMEM((_K, D), jnp.float32),
        pltpu.SemaphoreType.DMA,
        pltpu.SemaphoreType.DMA,
    ],
)
def _sc_gather(mem_hbm, ids_hbm, out_hbm, idx_v, buf0, buf1, sem0, sem1):
    wid = lax.axis_index("s") * _NC + lax.axis_index("c")
    base = wid * _G_BPW
    for c in range(_G_NCH):
        pltpu.sync_copy(ids_hbm.at[pl.ds(base + c * _K, _K)], idx_v.at[c])
    bufs = (buf0, buf1)
    sems = (sem0, sem1)
    cps = [None, None]
    for c in range(_G_NCH):
        cps[c % 2] = pltpu.async_copy(mem_hbm.at[idx_v.at[c]], bufs[c % 2], sems[c % 2])
        if c >= 1:
            cps[(c - 1) % 2].wait()
            pltpu.sync_copy(bufs[(c - 1) % 2], out_hbm.at[pl.ds(base + (c - 1) * _K, _K)])
    cps[(_G_NCH - 1) % 2].wait()
    pltpu.sync_copy(bufs[(_G_NCH - 1) % 2],
                    out_hbm.at[pl.ds(base + (_G_NCH - 1) * _K, _K)])


_R = 1024


def _gru_body(x_ref, h_ref, wi_ref, wh_ref, bi_ref, bh_ref, out_ref):
    h = h_ref[...]
    gi = jnp.dot(x_ref[...], wi_ref[...], preferred_element_type=jnp.float32) + bi_ref[...]
    gh = jnp.dot(h, wh_ref[...], preferred_element_type=jnp.float32) + bh_ref[...]
    r = jax.nn.sigmoid(gi[:, :D] + gh[:, :D])
    z = jax.nn.sigmoid(gi[:, D:2 * D] + gh[:, D:2 * D])
    n = jnp.tanh(gi[:, 2 * D:] + r * gh[:, 2 * D:])
    out_ref[...] = (1.0 - z) * n + z * h


_gru = pl.pallas_call(
    _gru_body,
    out_shape=jax.ShapeDtypeStruct((B, D), jnp.float32),
    grid=(B // _R,),
    in_specs=[
        pl.BlockSpec((_R, D), lambda i: (i, 0)),
        pl.BlockSpec((_R, D), lambda i: (i, 0)),
        pl.BlockSpec((D, 3 * D), lambda i: (0, 0)),
        pl.BlockSpec((D, 3 * D), lambda i: (0, 0)),
        pl.BlockSpec((1, 3 * D), lambda i: (0, 0)),
        pl.BlockSpec((1, 3 * D), lambda i: (0, 0)),
    ],
    out_specs=pl.BlockSpec((_R, D), lambda i: (i, 0)),
)


_S_NW = _NS
_S_BPW = B // _S_NW
_S_NCH = _S_BPW // _K
_S_ROWS = N_NODES // _S_NW
_CROWS = 128
_S_NCOPY = 49
_SLAB = _S_NCOPY * _CROWS
_LU_SZ = 6256

_scatter_mesh = plsc.VectorSubcoreMesh(
    core_axis_name="c", subcore_axis_name="s", num_cores=1)


@functools.partial(
    pl.kernel,
    out_type=[
        jax.ShapeDtypeStruct((N_NODES, D), jnp.float32),
        jax.ShapeDtypeStruct((N_NODES,), jnp.float32),
    ],
    mesh=_scatter_mesh,
    scratch_types=[
        pltpu.VMEM((_S_NCH, _K), jnp.int32),
        pltpu.VMEM((_S_NCH, _K), jnp.int32),
        pltpu.VMEM((_CROWS, D), jnp.float32),
        pltpu.VMEM((_CROWS, D), jnp.float32),
        pltpu.VMEM((_LU_SZ,), jnp.float32),
        pltpu.VMEM((_K, D), jnp.float32),
        pltpu.VMEM((_K,), jnp.float32),
        pltpu.SemaphoreType.DMA,
        pltpu.SemaphoreType.DMA,
        pltpu.SemaphoreType.DMA,
    ],
)
def _sc_scatter(mem_in, lu_in, ids_hbm, src_hbm, newh_hbm, ts_hbm,
                mem_out, lu_out,
                ids_v, src_v, cbuf0, cbuf1, lubuf, rbuf, tbuf,
                sem0, sem1, sem2):
    wid = lax.axis_index("s")
    base = wid * _S_BPW
    for c in range(_S_NCH):
        pltpu.sync_copy(ids_hbm.at[pl.ds(base + c * _K, _K)], ids_v.at[c])
        pltpu.sync_copy(src_hbm.at[pl.ds(base + c * _K, _K)], src_v.at[c])

    row0 = jnp.minimum((wid * _S_ROWS) // 8 * 8, N_NODES - _SLAB)

    pltpu.sync_copy(mem_in.at[pl.ds(row0, _SLAB)], mem_out.at[pl.ds(row0, _SLAB)])

    lu0 = jnp.minimum((wid * _S_ROWS) // 8 * 8, N_NODES - _LU_SZ)
    pltpu.sync_copy(lu_in.at[pl.ds(lu0, _LU_SZ)], lubuf)
    pltpu.sync_copy(lubuf, lu_out.at[pl.ds(lu0, _LU_SZ)])

    plsc.subcore_barrier()

    for c in range(_S_NCH):
        pltpu.async_copy(newh_hbm.at[src_v.at[c]], rbuf, sem0).wait()
        pltpu.sync_copy(rbuf, mem_out.at[ids_v.at[c]])
        pltpu.async_copy(ts_hbm.at[src_v.at[c]], tbuf, sem2).wait()
        pltpu.sync_copy(tbuf, lu_out.at[ids_v.at[c]])


def kernel(unique_node_ids, unique_messages, timestamps, memory, last_update,
           W_ih, W_hh, b_ih, b_hh):
    ids = unique_node_ids
    iota = jnp.arange(B, dtype=jnp.int32)
    aux = jnp.zeros((N_NODES,), jnp.int32).at[ids].max(iota)
    src = aux[ids]

    h = _sc_gather(memory, ids)
    new_h = _gru(unique_messages, h, W_ih.T, W_hh.T,
                 b_ih.reshape(1, 3 * D), b_hh.reshape(1, 3 * D))
    mem_out, lu_out = _sc_scatter(memory, last_update, ids, src, new_h,
                                  timestamps)
    return mem_out, lu_out


# speedup vs baseline: 11.2397x; 11.2397x over previous
import functools

import jax
import jax.numpy as jnp
from jax import lax
from jax.experimental import pallas as pl
from jax.experimental.pallas import tpu as pltpu
from jax.experimental.pallas import tpu_sc as plsc

N_NODES = 100000
D = 256
B = 16384

_NC = 2
_NS = 16
_NW = _NC * _NS
_K = 128

_G_BPW = B // _NW
_G_NCH = _G_BPW // _K

_gather_mesh = plsc.VectorSubcoreMesh(core_axis_name="c", subcore_axis_name="s")


@functools.partial(
    pl.kernel,
    out_type=jax.ShapeDtypeStruct((B, D), jnp.float32),
    mesh=_gather_mesh,
    scratch_types=[
        pltpu.VMEM((_G_NCH, _K), jnp.int32),
        pltpu.VMEM((_K, D), jnp.float32),
        pltpu.VMEM((_K, D), jnp.float32),
        pltpu.SemaphoreType.DMA,
        pltpu.SemaphoreType.DMA,
    ],
)
def _sc_gather(mem_hbm, ids_hbm, out_hbm, idx_v, buf0, buf1, sem0, sem1):
    wid = lax.axis_index("s") * _NC + lax.axis_index("c")
    base = wid * _G_BPW
    for c in range(_G_NCH):
        pltpu.sync_copy(ids_hbm.at[pl.ds(base + c * _K, _K)], idx_v.at[c])
    bufs = (buf0, buf1)
    sems = (sem0, sem1)
    cps = [None, None]
    for c in range(_G_NCH):
        cps[c % 2] = pltpu.async_copy(mem_hbm.at[idx_v.at[c]], bufs[c % 2], sems[c % 2])
        if c >= 1:
            cps[(c - 1) % 2].wait()
            pltpu.sync_copy(bufs[(c - 1) % 2], out_hbm.at[pl.ds(base + (c - 1) * _K, _K)])
    cps[(_G_NCH - 1) % 2].wait()
    pltpu.sync_copy(bufs[(_G_NCH - 1) % 2],
                    out_hbm.at[pl.ds(base + (_G_NCH - 1) * _K, _K)])


_R = 1024


def _gru_body(x_ref, h_ref, wi_ref, wh_ref, bi_ref, bh_ref, out_ref):
    h = h_ref[...]
    gi = jnp.dot(x_ref[...], wi_ref[...], preferred_element_type=jnp.float32) + bi_ref[...]
    gh = jnp.dot(h, wh_ref[...], preferred_element_type=jnp.float32) + bh_ref[...]
    r = jax.nn.sigmoid(gi[:, :D] + gh[:, :D])
    z = jax.nn.sigmoid(gi[:, D:2 * D] + gh[:, D:2 * D])
    n = jnp.tanh(gi[:, 2 * D:] + r * gh[:, 2 * D:])
    out_ref[...] = (1.0 - z) * n + z * h


_gru = pl.pallas_call(
    _gru_body,
    out_shape=jax.ShapeDtypeStruct((B, D), jnp.float32),
    grid=(B // _R,),
    in_specs=[
        pl.BlockSpec((_R, D), lambda i: (i, 0)),
        pl.BlockSpec((_R, D), lambda i: (i, 0)),
        pl.BlockSpec((D, 3 * D), lambda i: (0, 0)),
        pl.BlockSpec((D, 3 * D), lambda i: (0, 0)),
        pl.BlockSpec((1, 3 * D), lambda i: (0, 0)),
        pl.BlockSpec((1, 3 * D), lambda i: (0, 0)),
    ],
    out_specs=pl.BlockSpec((_R, D), lambda i: (i, 0)),
)


_S_NW = _NS
_S_BPW = B // _S_NW
_S_NCH = _S_BPW // _K
_S_ROWS = N_NODES // _S_NW
_CROWS = 128
_S_NCOPY = 49
_SLAB = _S_NCOPY * _CROWS
_LU_SZ = 6256

_scatter_mesh = plsc.VectorSubcoreMesh(
    core_axis_name="c", subcore_axis_name="s", num_cores=1)


@functools.partial(
    pl.kernel,
    out_type=[
        jax.ShapeDtypeStruct((N_NODES, D), jnp.float32),
        jax.ShapeDtypeStruct((N_NODES,), jnp.float32),
    ],
    mesh=_scatter_mesh,
    scratch_types=[
        pltpu.VMEM((_S_NCH, _K), jnp.int32),
        pltpu.VMEM((_S_NCH, _K), jnp.int32),
        pltpu.VMEM((_CROWS, D), jnp.float32),
        pltpu.VMEM((_CROWS, D), jnp.float32),
        pltpu.VMEM((_LU_SZ,), jnp.float32),
        pltpu.VMEM((_K,), jnp.float32),
        pltpu.SemaphoreType.DMA,
        pltpu.SemaphoreType.DMA,
        pltpu.SemaphoreType.DMA,
    ],
)
def _sc_scatter(mem_in, lu_in, ids_hbm, src_hbm, newh_hbm, ts_hbm,
                mem_out, lu_out,
                ids_v, src_v, cbuf0, cbuf1, lubuf, tbuf,
                sem0, sem1, sem2):
    wid = lax.axis_index("s")
    base = wid * _S_BPW
    for c in range(_S_NCH):
        pltpu.sync_copy(ids_hbm.at[pl.ds(base + c * _K, _K)], ids_v.at[c])
        pltpu.sync_copy(src_hbm.at[pl.ds(base + c * _K, _K)], src_v.at[c])

    row0 = jnp.minimum((wid * _S_ROWS) // 8 * 8, N_NODES - _SLAB)

    bufs = (cbuf0, cbuf1)
    st = [None, None]
    for i in range(_S_NCOPY):
        b = i % 2
        if i >= 2:
            st[b].wait()
        off = row0 + i * _CROWS
        pltpu.async_copy(mem_in.at[pl.ds(off, _CROWS)], bufs[b], sem0).wait()
        st[b] = pltpu.async_copy(bufs[b], mem_out.at[pl.ds(off, _CROWS)], sem1)
    st[(_S_NCOPY - 1) % 2].wait()
    st[(_S_NCOPY - 2) % 2].wait()

    lu0 = jnp.minimum((wid * _S_ROWS) // 8 * 8, N_NODES - _LU_SZ)
    pltpu.sync_copy(lu_in.at[pl.ds(lu0, _LU_SZ)], lubuf)
    pltpu.sync_copy(lubuf, lu_out.at[pl.ds(lu0, _LU_SZ)])

    plsc.subcore_barrier()

    st = [None, None]
    for c in range(_S_NCH):
        b = c % 2
        if c >= 2:
            st[b].wait()
        pltpu.async_copy(newh_hbm.at[src_v.at[c]], bufs[b], sem0).wait()
        st[b] = pltpu.async_copy(bufs[b], mem_out.at[ids_v.at[c]], sem1)
        pltpu.async_copy(ts_hbm.at[src_v.at[c]], tbuf, sem2).wait()
        pltpu.sync_copy(tbuf, lu_out.at[ids_v.at[c]])
    st[(_S_NCH - 1) % 2].wait()
    st[(_S_NCH - 2) % 2].wait()


def kernel(unique_node_ids, unique_messages, timestamps, memory, last_update,
           W_ih, W_hh, b_ih, b_hh):
    ids = unique_node_ids
    iota = jnp.arange(B, dtype=jnp.int32)
    aux = jnp.zeros((N_NODES,), jnp.int32).at[ids].max(iota)
    src = aux[ids]

    h = _sc_gather(memory, ids)
    new_h = _gru(unique_messages, h, W_ih.T, W_hh.T,
                 b_ih.reshape(1, 3 * D), b_hh.reshape(1, 3 * D))
    mem_out, lu_out = _sc_scatter(memory, last_update, ids, src, new_h,
                                  timestamps)
    return mem_out, lu_out
